# baseline (speedup 1.0000x reference)
import functools
import jax
import jax.numpy as jnp
from jax import lax
from jax.experimental import pallas as pl
from jax.experimental.pallas import tpu as pltpu
from jax.experimental.pallas import tpu_sc as plsc

N, C, H, W = 8, 5, 512, 512
NC, NS, L = 2, 16, 16
NW = NC * NS

TC_ROWS = 320

SC_ROWS = H - TC_ROWS
TILE_ROWS = SC_ROWS // 4
BLK_ROWS = 16
NBLK = TILE_ROWS // BLK_ROWS
VECS_PER_ROW = W // L

_K1 = 1 + (1 << 16)
_K2 = 2 + (4 << 16)


def _compute_vec(va, vb, vc, vt, accs, consts):
    a_p, a_t, a_i, a_m = accs
    zero, one, k1, k2 = consts
    gtb = vb > va
    mx = jnp.maximum(va, vb)
    gtc = vc > mx
    pp = jnp.where(gtc, k2, jnp.where(gtb, k1, zero))
    pt = vt + ((vt * vt) << 16)
    meq = pp == pt
    a_p = a_p + pp
    a_t = a_t + pt
    a_i = a_i + jnp.where(meq, pt, zero)
    a_m = a_m + jnp.where(meq, one, zero)
    return (a_p, a_t, a_i, a_m)


def _sc_partials(output, target):
    mesh = plsc.VectorSubcoreMesh(core_axis_name="c", subcore_axis_name="s")

    @functools.partial(
        pl.kernel,
        mesh=mesh,
        out_type=jax.ShapeDtypeStruct((NW, 4, L), jnp.int32),
        scratch_types=[
            pltpu.VMEM((2, 3, BLK_ROWS, W), jnp.float32),
            pltpu.VMEM((2, BLK_ROWS, W), jnp.int32),
            pltpu.VMEM((4, L), jnp.int32),
            pltpu.SemaphoreType.DMA,
            pltpu.SemaphoreType.DMA,
        ],
    )
    def k(out_hbm, tgt_hbm, res_hbm, xv, tv, ov, sem0, sem1):
        wid = lax.axis_index("s") * NC + lax.axis_index("c")
        n = wid // 4
        row0 = TC_ROWS + (wid % 4) * TILE_ROWS
        sems = (sem0, sem1)

        def issue(i):
            par = i % 2
            r0 = row0 + i * BLK_ROWS
            s = sems[par]
            return [
                pltpu.async_copy(
                    out_hbm.at[n, pl.ds(2, 3), pl.ds(r0, BLK_ROWS)],
                    xv.at[par], s),
                pltpu.async_copy(tgt_hbm.at[n, 2, pl.ds(r0, BLK_ROWS)],
                                 tv.at[par], s),
            ]

        zero = jnp.zeros((L,), jnp.int32)
        one = jnp.full((L,), 1, jnp.int32)
        k1 = jnp.full((L,), _K1, jnp.int32)
        k2 = jnp.full((L,), _K2, jnp.int32)
        consts = (zero, one, k1, k2)

        accs = (zero, zero, zero, zero)
        handles = issue(0)
        for i in range(NBLK):
            nxt = issue(i + 1) if i + 1 < NBLK else None
            for h in handles:
                h.wait()
            handles = nxt
            par = i % 2

            def body(j, accs, par=par):
                c0 = j * L
                for r in range(BLK_ROWS):
                    va = xv[par, 0, r, pl.ds(c0, L)]
                    vb = xv[par, 1, r, pl.ds(c0, L)]
                    vc = xv[par, 2, r, pl.ds(c0, L)]
                    vt = tv[par, r, pl.ds(c0, L)]
                    accs = _compute_vec(va, vb, vc, vt, accs, consts)
                return accs

            accs = lax.fori_loop(0, VECS_PER_ROW, body, accs)

        for row in range(4):
            ov[row, :] = accs[row]
        pltpu.sync_copy(ov, res_hbm.at[wid])

    return k(output, target)


def _tc_body(x_ref, t_ref, o_ref):
    a = x_ref[0, 2]
    b = x_ref[0, 3]
    c = x_ref[0, 4]
    t = t_ref[0, 0]
    gtb = b > a
    gtc = c > jnp.maximum(a, b)
    zero = jnp.int32(0)
    pp = jnp.where(gtc, jnp.int32(_K2), jnp.where(gtb, jnp.int32(_K1), zero))
    pt = t + ((t * t) << 16)
    meq = pp == pt
    vals = (pp, pt, jnp.where(meq, pt, zero), jnp.where(meq, jnp.int32(1), zero))
    for s, v in enumerate(vals):
        o_ref[0, s] = v.reshape(TC_ROWS // 8, 8, 4, 128).sum(axis=(0, 2))


def _tc_partials(output, target):
    return pl.pallas_call(
        _tc_body,
        grid=(N,),
        in_specs=[
            pl.BlockSpec((1, C, TC_ROWS, W), lambda n: (n, 0, 0, 0)),
            pl.BlockSpec((1, 1, TC_ROWS, W), lambda n: (n, 2, 0, 0)),
        ],
        out_specs=pl.BlockSpec((1, 4, 8, 128), lambda n: (n, 0, 0, 0)),
        out_shape=jax.ShapeDtypeStruct((N, 4, 8, 128), jnp.int32),
        compiler_params=pltpu.CompilerParams(
            dimension_semantics=("parallel",),
        ),
    )(output, target)


def _combine_body(sc_ref, tc_ref, o_ref):
    sc = sc_ref[...]
    slo = (sc & 0xFFFF).astype(jnp.float32)
    shi = (sc >> 16).astype(jnp.float32)
    slo = slo.sum(axis=2).reshape(N, 4, 4).sum(axis=1)
    shi = shi.sum(axis=2).reshape(N, 4, 4).sum(axis=1)
    tc = tc_ref[...]
    tlo = (tc & 0xFFFF).astype(jnp.float32).sum(axis=(2, 3))
    thi = (tc >> 16).astype(jnp.float32).sum(axis=(2, 3))
    lo = slo + tlo
    hi = shi + thi
    pi, pi2 = lo[:, 0], hi[:, 0]
    tt, tt2 = lo[:, 1], hi[:, 1]
    i1e, i2e = lo[:, 2], hi[:, 2]
    mq = lo[:, 3]
    m = jnp.float32(H * W)
    p2 = (pi2 - pi) * 0.5
    p1 = pi - 2.0 * p2
    p0 = m - p1 - p2
    t2c = (tt2 - tt) * 0.5
    t1c = tt - 2.0 * t2c
    t0c = m - t1c - t2c
    i2c = (i2e - i1e) * 0.5
    i1c = i1e - 2.0 * i2c
    i0c = mq - i1c - i2c
    eps = jnp.float32(1e-10)
    d0 = jnp.mean(2.0 * i0c / (p0 + t0c + eps))
    d1 = jnp.mean(2.0 * i1c / (p1 + t1c + eps))
    d2 = jnp.mean(2.0 * i2c / (p2 + t2c + eps))
    li = lax.broadcasted_iota(jnp.int32, (3,), 0)
    out = jnp.where(li == 0, d0, jnp.where(li == 1, d1, d2))
    o_ref[...] = out


def _combine(sc_parts, tc_parts):
    return pl.pallas_call(
        _combine_body,
        out_shape=jax.ShapeDtypeStruct((3,), jnp.float32),
    )(sc_parts, tc_parts)


def kernel(output, target):
    tgt = target.astype(jnp.int32)
    sc_parts = _sc_partials(output, tgt)
    tc_parts = _tc_partials(output, tgt)
    return _combine(sc_parts, tc_parts)


# speedup vs baseline: 1.0085x; 1.0085x over previous
import functools
import jax
import jax.numpy as jnp
from jax import lax
from jax.experimental import pallas as pl
from jax.experimental.pallas import tpu as pltpu
from jax.experimental.pallas import tpu_sc as plsc

N, C, H, W = 8, 5, 512, 512
NC, NS, L = 2, 16, 16
NW = NC * NS

TC_ROWS = 384
TC_HR = TC_ROWS // 2

SC_ROWS = H - TC_ROWS
TILE_ROWS = SC_ROWS // 4
BLK_ROWS = 16
NBLK = TILE_ROWS // BLK_ROWS
VECS_PER_ROW = W // L

_K1 = 1 + (1 << 16)
_K2 = 2 + (4 << 16)


def _compute_vec(va, vb, vc, vt, accs, consts):
    a_p, a_t, a_i, a_m = accs
    zero, one, k1, k2 = consts
    gtb = vb > va
    mx = jnp.maximum(va, vb)
    gtc = vc > mx
    pp = jnp.where(gtc, k2, jnp.where(gtb, k1, zero))
    pt = vt + ((vt * vt) << 16)
    meq = pp == pt
    a_p = a_p + pp
    a_t = a_t + pt
    a_i = a_i + jnp.where(meq, pt, zero)
    a_m = a_m + jnp.where(meq, one, zero)
    return (a_p, a_t, a_i, a_m)


def _sc_partials(output, target):
    mesh = plsc.VectorSubcoreMesh(core_axis_name="c", subcore_axis_name="s")

    @functools.partial(
        pl.kernel,
        mesh=mesh,
        out_type=jax.ShapeDtypeStruct((NW, 4, L), jnp.int32),
        scratch_types=[
            pltpu.VMEM((2, 3, BLK_ROWS, W), jnp.float32),
            pltpu.VMEM((2, BLK_ROWS, W), jnp.int32),
            pltpu.VMEM((4, L), jnp.int32),
            pltpu.SemaphoreType.DMA,
            pltpu.SemaphoreType.DMA,
        ],
    )
    def k(out_hbm, tgt_hbm, res_hbm, xv, tv, ov, sem0, sem1):
        wid = lax.axis_index("s") * NC + lax.axis_index("c")
        n = wid // 4
        row0 = TC_ROWS + (wid % 4) * TILE_ROWS
        sems = (sem0, sem1)

        def issue(i):
            par = i % 2
            r0 = row0 + i * BLK_ROWS
            s = sems[par]
            return [
                pltpu.async_copy(
                    out_hbm.at[n, pl.ds(2, 3), pl.ds(r0, BLK_ROWS)],
                    xv.at[par], s),
                pltpu.async_copy(tgt_hbm.at[n, 2, pl.ds(r0, BLK_ROWS)],
                                 tv.at[par], s),
            ]

        zero = jnp.zeros((L,), jnp.int32)
        one = jnp.full((L,), 1, jnp.int32)
        k1 = jnp.full((L,), _K1, jnp.int32)
        k2 = jnp.full((L,), _K2, jnp.int32)
        consts = (zero, one, k1, k2)

        accs = (zero, zero, zero, zero)
        handles = issue(0)
        for i in range(NBLK):
            nxt = issue(i + 1) if i + 1 < NBLK else None
            for h in handles:
                h.wait()
            handles = nxt
            par = i % 2

            def body(j, accs, par=par):
                c0 = j * L
                for r in range(BLK_ROWS):
                    va = xv[par, 0, r, pl.ds(c0, L)]
                    vb = xv[par, 1, r, pl.ds(c0, L)]
                    vc = xv[par, 2, r, pl.ds(c0, L)]
                    vt = tv[par, r, pl.ds(c0, L)]
                    accs = _compute_vec(va, vb, vc, vt, accs, consts)
                return accs

            accs = lax.fori_loop(0, VECS_PER_ROW, body, accs)

        for row in range(4):
            ov[row, :] = accs[row]
        pltpu.sync_copy(ov, res_hbm.at[wid])

    return k(output, target)


def _tc_half(a, b, c, t):
    gtb = b > a
    gtc = c > jnp.maximum(a, b)
    zero = jnp.int32(0)
    pp = jnp.where(gtc, jnp.int32(_K2), jnp.where(gtb, jnp.int32(_K1), zero))
    pt = t + ((t * t) << 16)
    meq = pp == pt
    vals = (pp, pt, jnp.where(meq, pt, zero), jnp.where(meq, jnp.int32(1), zero))
    return [v.reshape(TC_HR // 8, 8, 4, 128).sum(axis=(0, 2)) for v in vals]


def _tc_body(a1, a2, b1, b2, c1, c2, t1, t2, o_ref):
    h1 = _tc_half(a1[0, 0], b1[0, 0], c1[0, 0], t1[0, 0])
    h2 = _tc_half(a2[0, 0], b2[0, 0], c2[0, 0], t2[0, 0])
    for s in range(4):
        o_ref[0, s] = h1[s] + h2[s]


def _tc_partials(output, target):
    def spec(ch, h):
        return pl.BlockSpec((1, 1, TC_HR, W), lambda n, ch=ch, h=h: (n, ch, h, 0))

    return pl.pallas_call(
        _tc_body,
        grid=(N,),
        in_specs=[
            spec(2, 0), spec(2, 1),
            spec(3, 0), spec(3, 1),
            spec(4, 0), spec(4, 1),
            spec(2, 0), spec(2, 1),
        ],
        out_specs=pl.BlockSpec((1, 4, 8, 128), lambda n: (n, 0, 0, 0)),
        out_shape=jax.ShapeDtypeStruct((N, 4, 8, 128), jnp.int32),
        compiler_params=pltpu.CompilerParams(
            dimension_semantics=("parallel",),
        ),
    )(output, output, output, output, output, output, target, target)


def _combine_body(sc_ref, tc_ref, o_ref):
    sc = sc_ref[...]
    slo = (sc & 0xFFFF).astype(jnp.float32)
    shi = (sc >> 16).astype(jnp.float32)
    slo = slo.sum(axis=2).reshape(N, 4, 4).sum(axis=1)
    shi = shi.sum(axis=2).reshape(N, 4, 4).sum(axis=1)
    tc = tc_ref[...]
    tlo = (tc & 0xFFFF).astype(jnp.float32).sum(axis=(2, 3))
    thi = (tc >> 16).astype(jnp.float32).sum(axis=(2, 3))
    lo = slo + tlo
    hi = shi + thi
    pi, pi2 = lo[:, 0], hi[:, 0]
    tt, tt2 = lo[:, 1], hi[:, 1]
    i1e, i2e = lo[:, 2], hi[:, 2]
    mq = lo[:, 3]
    m = jnp.float32(H * W)
    p2 = (pi2 - pi) * 0.5
    p1 = pi - 2.0 * p2
    p0 = m - p1 - p2
    t2c = (tt2 - tt) * 0.5
    t1c = tt - 2.0 * t2c
    t0c = m - t1c - t2c
    i2c = (i2e - i1e) * 0.5
    i1c = i1e - 2.0 * i2c
    i0c = mq - i1c - i2c
    eps = jnp.float32(1e-10)
    d0 = jnp.mean(2.0 * i0c / (p0 + t0c + eps))
    d1 = jnp.mean(2.0 * i1c / (p1 + t1c + eps))
    d2 = jnp.mean(2.0 * i2c / (p2 + t2c + eps))
    li = lax.broadcasted_iota(jnp.int32, (3,), 0)
    out = jnp.where(li == 0, d0, jnp.where(li == 1, d1, d2))
    o_ref[...] = out


def _combine(sc_parts, tc_parts):
    return pl.pallas_call(
        _combine_body,
        out_shape=jax.ShapeDtypeStruct((3,), jnp.float32),
    )(sc_parts, tc_parts)


def kernel(output, target):
    tgt = target.astype(jnp.int32)
    sc_parts = _sc_partials(output, tgt)
    tc_parts = _tc_partials(output, tgt)
    return _combine(sc_parts, tc_parts)


# speedup vs baseline: 1.0539x; 1.0450x over previous
import functools
import jax
import jax.numpy as jnp
from jax import lax
from jax.experimental import pallas as pl
from jax.experimental.pallas import tpu as pltpu
from jax.experimental.pallas import tpu_sc as plsc

N, C, H, W = 8, 5, 512, 512
NC, NS, L = 2, 16, 16
NW = NC * NS

TC_ROWS = 384
TC_RB = 128
TC_CHUNKS_PER_N = TC_ROWS // TC_RB
TC_NCHUNKS = N * TC_CHUNKS_PER_N
TC_NB = 4
TC_AHEAD = 3

SC_ROWS = H - TC_ROWS
TILE_ROWS = SC_ROWS // 4
BLK_ROWS = 16
NBLK = TILE_ROWS // BLK_ROWS
VECS_PER_ROW = W // L

_K1 = 1 + (1 << 16)
_K2 = 2 + (4 << 16)


def _compute_vec(va, vb, vc, vt, accs, consts):
    a_p, a_t, a_i, a_m = accs
    zero, one, k1, k2 = consts
    gtb = vb > va
    mx = jnp.maximum(va, vb)
    gtc = vc > mx
    pp = jnp.where(gtc, k2, jnp.where(gtb, k1, zero))
    pt = vt + ((vt * vt) << 16)
    meq = pp == pt
    a_p = a_p + pp
    a_t = a_t + pt
    a_i = a_i + jnp.where(meq, pt, zero)
    a_m = a_m + jnp.where(meq, one, zero)
    return (a_p, a_t, a_i, a_m)


def _sc_partials(output, target):
    mesh = plsc.VectorSubcoreMesh(core_axis_name="c", subcore_axis_name="s")

    @functools.partial(
        pl.kernel,
        mesh=mesh,
        out_type=jax.ShapeDtypeStruct((NW, 4, L), jnp.int32),
        scratch_types=[
            pltpu.VMEM((2, 3, BLK_ROWS, W), jnp.float32),
            pltpu.VMEM((2, BLK_ROWS, W), jnp.int32),
            pltpu.VMEM((4, L), jnp.int32),
            pltpu.SemaphoreType.DMA,
            pltpu.SemaphoreType.DMA,
        ],
    )
    def k(out_hbm, tgt_hbm, res_hbm, xv, tv, ov, sem0, sem1):
        wid = lax.axis_index("s") * NC + lax.axis_index("c")
        n = wid // 4
        row0 = TC_ROWS + (wid % 4) * TILE_ROWS
        sems = (sem0, sem1)

        def issue(i):
            par = i % 2
            r0 = row0 + i * BLK_ROWS
            s = sems[par]
            return [
                pltpu.async_copy(
                    out_hbm.at[n, pl.ds(2, 3), pl.ds(r0, BLK_ROWS)],
                    xv.at[par], s),
                pltpu.async_copy(tgt_hbm.at[n, 2, pl.ds(r0, BLK_ROWS)],
                                 tv.at[par], s),
            ]

        zero = jnp.zeros((L,), jnp.int32)
        one = jnp.full((L,), 1, jnp.int32)
        k1 = jnp.full((L,), _K1, jnp.int32)
        k2 = jnp.full((L,), _K2, jnp.int32)
        consts = (zero, one, k1, k2)

        accs = (zero, zero, zero, zero)
        handles = issue(0)
        for i in range(NBLK):
            nxt = issue(i + 1) if i + 1 < NBLK else None
            for h in handles:
                h.wait()
            handles = nxt
            par = i % 2

            def body(j, accs, par=par):
                c0 = j * L
                for r in range(BLK_ROWS):
                    va = xv[par, 0, r, pl.ds(c0, L)]
                    vb = xv[par, 1, r, pl.ds(c0, L)]
                    vc = xv[par, 2, r, pl.ds(c0, L)]
                    vt = tv[par, r, pl.ds(c0, L)]
                    accs = _compute_vec(va, vb, vc, vt, accs, consts)
                return accs

            accs = lax.fori_loop(0, VECS_PER_ROW, body, accs)

        for row in range(4):
            ov[row, :] = accs[row]
        pltpu.sync_copy(ov, res_hbm.at[wid])

    return k(output, target)


def _tc_chunk_vals(a, b, c, t):
    gtb = b > a
    gtc = c > jnp.maximum(a, b)
    zero = jnp.int32(0)
    pp = jnp.where(gtc, jnp.int32(_K2), jnp.where(gtb, jnp.int32(_K1), zero))
    pt = t + ((t * t) << 16)
    meq = pp == pt
    vals = (pp, pt, jnp.where(meq, pt, zero), jnp.where(meq, jnp.int32(1), zero))
    return [v.reshape(TC_RB // 8, 8, 4, 128).sum(axis=(0, 2)) for v in vals]


def _tc_body(x_hbm, t_hbm, o_ref, xb, tb, sem0, sem1, sem2, sem3):
    sems = (sem0, sem1, sem2, sem3)

    def issue(i):
        n, ci = divmod(i, TC_CHUNKS_PER_N)
        r0 = ci * TC_RB
        slot = i % TC_NB
        s = sems[slot]
        return [
            pltpu.async_copy(x_hbm.at[n, 2, pl.ds(r0, TC_RB)],
                             xb.at[slot, 0], s),
            pltpu.async_copy(x_hbm.at[n, 3, pl.ds(r0, TC_RB)],
                             xb.at[slot, 1], s),
            pltpu.async_copy(x_hbm.at[n, 4, pl.ds(r0, TC_RB)],
                             xb.at[slot, 2], s),
            pltpu.async_copy(t_hbm.at[n, 2, pl.ds(r0, TC_RB)],
                             tb.at[slot], s),
        ]

    handles = {}
    for i in range(TC_AHEAD):
        handles[i] = issue(i)
    accs = None
    for i in range(TC_NCHUNKS):
        if i + TC_AHEAD < TC_NCHUNKS:
            handles[i + TC_AHEAD] = issue(i + TC_AHEAD)
        for h in handles.pop(i):
            h.wait()
        slot = i % TC_NB
        n, ci = divmod(i, TC_CHUNKS_PER_N)
        vals = _tc_chunk_vals(xb[slot, 0], xb[slot, 1], xb[slot, 2], tb[slot])
        if ci == 0:
            accs = vals
        else:
            accs = [a + v for a, v in zip(accs, vals)]
        if ci == TC_CHUNKS_PER_N - 1:
            for s in range(4):
                o_ref[n, s] = accs[s]


def _tc_partials(output, target):
    return pl.pallas_call(
        _tc_body,
        in_specs=[
            pl.BlockSpec(memory_space=pltpu.MemorySpace.HBM),
            pl.BlockSpec(memory_space=pltpu.MemorySpace.HBM),
        ],
        out_shape=jax.ShapeDtypeStruct((N, 4, 8, 128), jnp.int32),
        scratch_shapes=[
            pltpu.VMEM((TC_NB, 3, TC_RB, W), jnp.float32),
            pltpu.VMEM((TC_NB, TC_RB, W), jnp.int32),
            pltpu.SemaphoreType.DMA,
            pltpu.SemaphoreType.DMA,
            pltpu.SemaphoreType.DMA,
            pltpu.SemaphoreType.DMA,
        ],
    )(output, target)


def _combine_body(sc_ref, tc_ref, o_ref):
    sc = sc_ref[...]
    slo = (sc & 0xFFFF).astype(jnp.float32)
    shi = (sc >> 16).astype(jnp.float32)
    slo = slo.sum(axis=2).reshape(N, 4, 4).sum(axis=1)
    shi = shi.sum(axis=2).reshape(N, 4, 4).sum(axis=1)
    tc = tc_ref[...]
    tlo = (tc & 0xFFFF).astype(jnp.float32).sum(axis=(2, 3))
    thi = (tc >> 16).astype(jnp.float32).sum(axis=(2, 3))
    lo = slo + tlo
    hi = shi + thi
    pi, pi2 = lo[:, 0], hi[:, 0]
    tt, tt2 = lo[:, 1], hi[:, 1]
    i1e, i2e = lo[:, 2], hi[:, 2]
    mq = lo[:, 3]
    m = jnp.float32(H * W)
    p2 = (pi2 - pi) * 0.5
    p1 = pi - 2.0 * p2
    p0 = m - p1 - p2
    t2c = (tt2 - tt) * 0.5
    t1c = tt - 2.0 * t2c
    t0c = m - t1c - t2c
    i2c = (i2e - i1e) * 0.5
    i1c = i1e - 2.0 * i2c
    i0c = mq - i1c - i2c
    eps = jnp.float32(1e-10)
    d0 = jnp.mean(2.0 * i0c / (p0 + t0c + eps))
    d1 = jnp.mean(2.0 * i1c / (p1 + t1c + eps))
    d2 = jnp.mean(2.0 * i2c / (p2 + t2c + eps))
    li = lax.broadcasted_iota(jnp.int32, (3,), 0)
    out = jnp.where(li == 0, d0, jnp.where(li == 1, d1, d2))
    o_ref[...] = out


def _combine(sc_parts, tc_parts):
    return pl.pallas_call(
        _combine_body,
        out_shape=jax.ShapeDtypeStruct((3,), jnp.float32),
    )(sc_parts, tc_parts)


def kernel(output, target):
    tgt = target.astype(jnp.int32)
    sc_parts = _sc_partials(output, tgt)
    tc_parts = _tc_partials(output, tgt)
    return _combine(sc_parts, tc_parts)
